# baseline (speedup 1.0000x reference)
import jax
import jax.numpy as jnp
import numpy as np
from jax import lax
from jax.experimental import pallas as pl
from jax.experimental.pallas import tpu as pltpu
from jax.experimental.pallas import tpu_sc as plsc

VOCAB = 100000
D_MODEL = 768
MAX_LEN = 8192
BATCH = 4
SEQ = 2048

NUM_CORES = 2
NUM_SUBCORES = 16
NUM_WORKERS = NUM_CORES * NUM_SUBCORES
S_PER_W = SEQ // NUM_WORKERS
LANES = 16
PAIRS = D_MODEL // (2 * LANES)

CHUNK = 32
NBUF = 4
CPB = S_PER_W // CHUNK
NCH = BATCH * CPB


def _pos_encoding_np(max_len, d_model):
    pos = np.arange(max_len, dtype=np.float32)[:, None]
    i = np.arange(d_model, dtype=np.float32)[None, :]
    angle_rates = 1.0 / np.power(10000.0, (2.0 * np.floor(i / 2.0)) / d_model)
    angles = pos * angle_rates
    pe = np.zeros((max_len, d_model), dtype=np.float32)
    pe[:, 0::2] = np.sin(angles[:, 0::2])
    pe[:, 1::2] = np.cos(angles[:, 1::2])
    return pe


def _packed_pe_np():
    pe = _pos_encoding_np(SEQ, D_MODEL)
    pe_r = pe.reshape(SEQ, PAIRS, 2, LANES)
    return pe_r.transpose(0, 1, 3, 2).reshape(SEQ, D_MODEL)


_PE_PACKED = _packed_pe_np()


def _emb_kernel(x_hbm, table_hbm, pe_hbm, out_hbm,
                i0, i1, i2, i3, b0, b1, b2, b3, pe_v,
                isem, g0, g1, g2, g3, w0, w1, w2, w3):
    wid = lax.axis_index("s") * NUM_CORES + lax.axis_index("c")
    s0 = wid * S_PER_W

    idxs = (i0, i1, i2, i3)
    icopies = [
        pltpu.async_copy(x_hbm.at[b, pl.ds(s0, S_PER_W)], idxs[b], isem)
        for b in range(BATCH)
    ]
    for c in icopies:
        c.wait()

    bufs = (b0, b1, b2, b3)
    gsems = (g0, g1, g2, g3)
    wsems = (w0, w1, w2, w3)
    gathers = [None] * NBUF
    writes = [None] * NBUF

    def start_gather(c):
        b, h = divmod(c, CPB)
        bi = c % NBUF
        idx = idxs[b].at[pl.ds(h * CHUNK, CHUNK)]
        gathers[bi] = pltpu.async_copy(
            table_hbm.at[idx], bufs[bi], gsems[bi])

    for c in range(NBUF):
        start_gather(c)

    pltpu.sync_copy(
        pe_hbm.at[pl.ds(s0 * (D_MODEL // 2), S_PER_W * (D_MODEL // 2))], pe_v)

    for c in range(NCH):
        bi = c % NBUF
        if c >= 1 and (c - 1) + NBUF < NCH:
            pb = (c - 1) % NBUF
            writes[pb].wait()
            writes[pb] = None
            start_gather((c - 1) + NBUF)

        b, h = divmod(c, CPB)
        buf = bufs[bi]
        gathers[bi].wait()

        @plsc.parallel_loop(0, CHUNK, unroll=2)
        def add_row(t, buf=buf, h=h):
            off = pl.multiple_of(
                (h * CHUNK + t) * (D_MODEL // 2), D_MODEL // 2)
            for p in range(PAIRS):
                w = pe_v[pl.ds(off + p * LANES, LANES)]
                lo = lax.bitcast_convert_type(w << 16, jnp.float32)
                hi = lax.bitcast_convert_type(w & jnp.int32(-65536), jnp.float32)
                plsc.addupdate(buf.at[t, pl.ds(p * 2 * LANES, LANES)], lo)
                plsc.addupdate(buf.at[t, pl.ds(p * 2 * LANES + LANES, LANES)], hi)

        writes[bi] = pltpu.async_copy(
            buf, out_hbm.at[b, pl.ds(s0 + h * CHUNK, CHUNK), :], wsems[bi])

    for wr in writes:
        if wr is not None:
            wr.wait()


@jax.jit
def kernel(x, tok_table):
    mesh = plsc.VectorSubcoreMesh(core_axis_name="c", subcore_axis_name="s")
    call = pl.kernel(
        _emb_kernel,
        out_type=jax.ShapeDtypeStruct((BATCH, SEQ, D_MODEL), jnp.float32),
        mesh=mesh,
        scratch_types=[
            pltpu.VMEM((S_PER_W,), jnp.int32),
            pltpu.VMEM((S_PER_W,), jnp.int32),
            pltpu.VMEM((S_PER_W,), jnp.int32),
            pltpu.VMEM((S_PER_W,), jnp.int32),
            pltpu.VMEM((CHUNK, D_MODEL), jnp.float32),
            pltpu.VMEM((CHUNK, D_MODEL), jnp.float32),
            pltpu.VMEM((CHUNK, D_MODEL), jnp.float32),
            pltpu.VMEM((CHUNK, D_MODEL), jnp.float32),
            pltpu.VMEM((S_PER_W * D_MODEL // 2,), jnp.int32),
            pltpu.SemaphoreType.DMA,
            pltpu.SemaphoreType.DMA,
            pltpu.SemaphoreType.DMA,
            pltpu.SemaphoreType.DMA,
            pltpu.SemaphoreType.DMA,
            pltpu.SemaphoreType.DMA,
            pltpu.SemaphoreType.DMA,
            pltpu.SemaphoreType.DMA,
            pltpu.SemaphoreType.DMA,
        ],
    )
    pe_bf = jnp.asarray(_PE_PACKED).astype(jnp.bfloat16)
    pe_words = jax.lax.bitcast_convert_type(
        pe_bf.reshape(SEQ * D_MODEL // 2, 2), jnp.int32)
    return call(x, tok_table, pe_words)


# speedup vs baseline: 1.3304x; 1.3304x over previous
import jax
import jax.numpy as jnp
import numpy as np
from jax import lax
from jax.experimental import pallas as pl
from jax.experimental.pallas import tpu as pltpu
from jax.experimental.pallas import tpu_sc as plsc

VOCAB = 100000
D_MODEL = 768
MAX_LEN = 8192
BATCH = 4
SEQ = 2048

NUM_CORES = 2
NUM_SUBCORES = 16
NUM_WORKERS = NUM_CORES * NUM_SUBCORES
S_PER_W = SEQ // NUM_WORKERS
LANES = 16
PAIRS = D_MODEL // (2 * LANES)

CHUNK = 32
NBUF = 4
CPB = S_PER_W // CHUNK
NCH = BATCH * CPB


def _pos_encoding_np(max_len, d_model):
    pos = np.arange(max_len, dtype=np.float32)[:, None]
    i = np.arange(d_model, dtype=np.float32)[None, :]
    angle_rates = 1.0 / np.power(10000.0, (2.0 * np.floor(i / 2.0)) / d_model)
    angles = pos * angle_rates
    pe = np.zeros((max_len, d_model), dtype=np.float32)
    pe[:, 0::2] = np.sin(angles[:, 0::2])
    pe[:, 1::2] = np.cos(angles[:, 1::2])
    return pe


def _packed_pe_np():
    pe = _pos_encoding_np(SEQ, D_MODEL)
    pe_r = pe.reshape(SEQ, PAIRS, 2, LANES)
    return pe_r.transpose(0, 1, 3, 2).reshape(SEQ, D_MODEL)


_PE_PACKED = _packed_pe_np()


def _emb_kernel(x_hbm, table_hbm, pe_hbm, out_hbm, idx_v, rows_v, pe_v, sem):
    wid = lax.axis_index("s") * NUM_CORES + lax.axis_index("c")
    s0 = wid * S_PER_W

    pltpu.sync_copy(pe_hbm.at[pl.ds(s0, S_PER_W), :], pe_v)

    for b in range(BATCH):
        pltpu.sync_copy(x_hbm.at[b, pl.ds(s0, S_PER_W)], idx_v)
        pltpu.async_copy(table_hbm.at[idx_v], rows_v, sem).wait()

        def add_row(t, _):
            for g in range(D_MODEL // LANES):
                sl = pl.ds(g * LANES, LANES)
                rows_v[t, sl] = rows_v[t, sl] + pe_v[t, sl]
            return _

        lax.fori_loop(0, S_PER_W, add_row, 0)

        pltpu.sync_copy(rows_v, out_hbm.at[b, pl.ds(s0, S_PER_W), :])


@jax.jit
def kernel(x, tok_table):
    mesh = plsc.VectorSubcoreMesh(core_axis_name="c", subcore_axis_name="s")
    call = pl.kernel(
        _emb_kernel,
        out_type=jax.ShapeDtypeStruct((BATCH, SEQ, D_MODEL), jnp.float32),
        mesh=mesh,
        scratch_types=[
            pltpu.VMEM((S_PER_W,), jnp.int32),
            pltpu.VMEM((S_PER_W, D_MODEL), jnp.float32),
            pltpu.VMEM((S_PER_W, D_MODEL), jnp.float32),
            pltpu.SemaphoreType.DMA,
        ],
    )
    return call(x, tok_table, jnp.asarray(_pos_encoding_np(SEQ, D_MODEL)))


# speedup vs baseline: 1.6934x; 1.2728x over previous
import jax
import jax.numpy as jnp
import numpy as np
from jax import lax
from jax.experimental import pallas as pl
from jax.experimental.pallas import tpu as pltpu
from jax.experimental.pallas import tpu_sc as plsc

VOCAB = 100000
D_MODEL = 768
MAX_LEN = 8192
BATCH = 4
SEQ = 2048

NUM_CORES = 2
NUM_SUBCORES = 16
NUM_WORKERS = NUM_CORES * NUM_SUBCORES
S_PER_W = SEQ // NUM_WORKERS
LANES = 16
PAIRS = D_MODEL // (2 * LANES)

CHUNK = 32
NBUF = 4
CPB = S_PER_W // CHUNK
NCH = BATCH * CPB


def _pos_encoding_np(max_len, d_model):
    pos = np.arange(max_len, dtype=np.float32)[:, None]
    i = np.arange(d_model, dtype=np.float32)[None, :]
    angle_rates = 1.0 / np.power(10000.0, (2.0 * np.floor(i / 2.0)) / d_model)
    angles = pos * angle_rates
    pe = np.zeros((max_len, d_model), dtype=np.float32)
    pe[:, 0::2] = np.sin(angles[:, 0::2])
    pe[:, 1::2] = np.cos(angles[:, 1::2])
    return pe


def _packed_pe_np():
    pe = _pos_encoding_np(SEQ, D_MODEL)
    pe_r = pe.reshape(SEQ, PAIRS, 2, LANES)
    return pe_r.transpose(0, 1, 3, 2).reshape(SEQ, D_MODEL)


_PE_PACKED = _packed_pe_np()


def _emb_kernel(x_hbm, table_hbm, pe_hbm, out_hbm,
                idx_v, rows0, rows1, pe_v, g0, g1, w0, w1):
    wid = lax.axis_index("s") * NUM_CORES + lax.axis_index("c")
    s0 = wid * S_PER_W

    idxs = (idx_v.at[0], idx_v.at[1], idx_v.at[2], idx_v.at[3])
    for b in range(BATCH):
        pltpu.sync_copy(x_hbm.at[b, pl.ds(s0, S_PER_W)], idxs[b])

    bufs = (rows0, rows1)
    gsems = (g0, g1)
    wsems = (w0, w1)
    gathers = [None, None]
    writes = [None, None]

    def start_gather(k):
        gathers[k % 2] = pltpu.async_copy(
            table_hbm.at[idxs[k]], bufs[k % 2], gsems[k % 2])

    start_gather(0)
    pltpu.sync_copy(pe_hbm.at[pl.ds(s0, S_PER_W), :], pe_v)

    for k in range(BATCH):
        buf = bufs[k % 2]
        gathers[k % 2].wait()

        if k + 1 < BATCH:
            nxt = (k + 1) % 2
            if writes[nxt] is not None:
                writes[nxt].wait()
                writes[nxt] = None
            start_gather(k + 1)

        @plsc.parallel_loop(0, S_PER_W, unroll=2)
        def add_row(t, buf=buf):
            for p in range(PAIRS):
                w = pe_v[t, pl.ds(p * LANES, LANES)]
                lo = lax.bitcast_convert_type(w << 16, jnp.float32)
                hi = lax.bitcast_convert_type(w & jnp.int32(-65536), jnp.float32)
                plsc.addupdate(buf.at[t, pl.ds(p * 2 * LANES, LANES)], lo)
                plsc.addupdate(buf.at[t, pl.ds(p * 2 * LANES + LANES, LANES)], hi)

        writes[k % 2] = pltpu.async_copy(
            buf, out_hbm.at[k, pl.ds(s0, S_PER_W), :], wsems[k % 2])

    for wr in writes:
        if wr is not None:
            wr.wait()


@jax.jit
def kernel(x, tok_table):
    mesh = plsc.VectorSubcoreMesh(core_axis_name="c", subcore_axis_name="s")
    call = pl.kernel(
        _emb_kernel,
        out_type=jax.ShapeDtypeStruct((BATCH, SEQ, D_MODEL), jnp.float32),
        mesh=mesh,
        scratch_types=[
            pltpu.VMEM((BATCH, S_PER_W), jnp.int32),
            pltpu.VMEM((S_PER_W, D_MODEL), jnp.float32),
            pltpu.VMEM((S_PER_W, D_MODEL), jnp.float32),
            pltpu.VMEM((S_PER_W, D_MODEL // 2), jnp.int32),
            pltpu.SemaphoreType.DMA,
            pltpu.SemaphoreType.DMA,
            pltpu.SemaphoreType.DMA,
            pltpu.SemaphoreType.DMA,
        ],
    )
    pe_bf = jnp.asarray(_PE_PACKED).astype(jnp.bfloat16)
    pe_words = jax.lax.bitcast_convert_type(
        pe_bf.reshape(SEQ, D_MODEL // 2, 2), jnp.int32)
    return call(x, tok_table, pe_words)
